# baseline (speedup 1.0000x reference)
import jax
import jax.numpy as jnp
from jax.experimental import pallas as pl
from jax.experimental.pallas import tpu as pltpu

_NCHUNKS = 8


def _copy_dma(in_ref, out_ref, sems):
    rows = in_ref.shape[0]
    chunk = rows // _NCHUNKS
    for i in range(_NCHUNKS):
        sl = pl.ds(i * chunk, chunk)
        pltpu.make_async_copy(in_ref.at[sl], out_ref.at[sl], sems.at[i]).start()
    for i in range(_NCHUNKS):
        sl = pl.ds(i * chunk, chunk)
        pltpu.make_async_copy(in_ref.at[sl], out_ref.at[sl], sems.at[i]).wait()


def kernel(x):
    flat = x.reshape(-1, x.shape[-1])
    out = pl.pallas_call(
        _copy_dma,
        in_specs=[pl.BlockSpec(memory_space=pl.ANY)],
        out_specs=pl.BlockSpec(memory_space=pl.ANY),
        out_shape=jax.ShapeDtypeStruct(flat.shape, flat.dtype),
        scratch_shapes=[pltpu.SemaphoreType.DMA((_NCHUNKS,))],
    )(flat)
    return out.reshape(x.shape)


# speedup vs baseline: 43.9382x; 43.9382x over previous
import jax
import jax.numpy as jnp
from jax.experimental import pallas as pl


def _copy_block(in_ref, out_ref):
    out_ref[...] = in_ref[...]


def kernel(x):
    flat = x.reshape(-1, x.shape[-1])
    m, n = flat.shape
    bm = 4096
    out = pl.pallas_call(
        _copy_block,
        grid=(m // bm,),
        in_specs=[pl.BlockSpec((bm, n), lambda i: (i, 0))],
        out_specs=pl.BlockSpec((bm, n), lambda i: (i, 0)),
        out_shape=jax.ShapeDtypeStruct((m, n), flat.dtype),
    )(flat)
    return out.reshape(x.shape)


# speedup vs baseline: 48.9585x; 1.1143x over previous
import jax
import jax.numpy as jnp
from jax.experimental import pallas as pl


def _copy_block(in_ref, out_ref):
    out_ref[...] = in_ref[...]


def kernel(x):
    flat = x.reshape(-1, x.shape[-1])
    m, n = flat.shape
    bm = 20480
    out = pl.pallas_call(
        _copy_block,
        grid=(m // bm,),
        in_specs=[pl.BlockSpec((bm, n), lambda i: (i, 0))],
        out_specs=pl.BlockSpec((bm, n), lambda i: (i, 0)),
        out_shape=jax.ShapeDtypeStruct((m, n), flat.dtype),
    )(flat)
    return out.reshape(x.shape)


# speedup vs baseline: 49.1993x; 1.0049x over previous
import jax
import jax.numpy as jnp
from jax.experimental import pallas as pl


def _copy_block(in_ref, out_ref):
    out_ref[...] = in_ref[...]


def kernel(x):
    flat = x.reshape(-1, x.shape[-1])
    m, n = flat.shape
    bm = 25600
    out = pl.pallas_call(
        _copy_block,
        grid=(m // bm,),
        in_specs=[pl.BlockSpec((bm, n), lambda i: (i, 0))],
        out_specs=pl.BlockSpec((bm, n), lambda i: (i, 0)),
        out_shape=jax.ShapeDtypeStruct((m, n), flat.dtype),
    )(flat)
    return out.reshape(x.shape)


# speedup vs baseline: 49.5655x; 1.0074x over previous
import jax
import jax.numpy as jnp
from jax.experimental import pallas as pl


def _copy_block(in_ref, out_ref):
    out_ref[...] = in_ref[...]


def kernel(x):
    flat = x.reshape(-1, x.shape[-1])
    m, n = flat.shape
    bm = 28672
    out = pl.pallas_call(
        _copy_block,
        grid=(pl.cdiv(m, bm),),
        in_specs=[pl.BlockSpec((bm, n), lambda i: (i, 0))],
        out_specs=pl.BlockSpec((bm, n), lambda i: (i, 0)),
        out_shape=jax.ShapeDtypeStruct((m, n), flat.dtype),
    )(flat)
    return out.reshape(x.shape)
